# baseline (speedup 1.0000x reference)
import functools

import jax
import jax.numpy as jnp
from jax import lax
from jax.experimental import pallas as pl
from jax.experimental.pallas import tpu as pltpu
from jax.experimental.pallas import tpu_sc as plsc

NC = 2
NS = 16
K = 128
NPAD = 10240
DW = 128


def _sc_degree(dst2d, n):
    nchunks = dst2d.shape[0]
    cpt = nchunks // (NC * NS)
    rpt = NPAD // NS
    ones = jnp.ones((K, DW), jnp.float32)
    zeros = jnp.zeros((rpt, DW), jnp.float32)
    mesh = plsc.VectorSubcoreMesh(core_axis_name="c", subcore_axis_name="s")

    @functools.partial(
        pl.kernel,
        out_type=jax.ShapeDtypeStruct((NC, NPAD, DW), jnp.float32),
        mesh=mesh,
        scratch_types=[
            pltpu.VMEM((cpt, K), jnp.int32),
            pltpu.VMEM((K, DW), jnp.float32),
            pltpu.VMEM_SHARED((NPAD, DW), jnp.float32),
        ],
    )
    def deg_kernel(dst_hbm, ones_hbm, z_hbm, out_hbm, dst_v, ones_v, acc):
        c = lax.axis_index("c")
        s = lax.axis_index("s")
        wid = c * NS + s
        pltpu.sync_copy(z_hbm, acc.at[pl.ds(s * rpt, rpt)])
        pltpu.sync_copy(dst_hbm.at[pl.ds(wid * cpt, cpt)], dst_v)
        pltpu.sync_copy(ones_hbm, ones_v)
        plsc.subcore_barrier()

        def body(j, carry):
            pltpu.sync_copy(ones_v, acc.at[dst_v.at[j]], add=True)
            return carry

        lax.fori_loop(0, cpt, body, 0, unroll=False)
        plsc.subcore_barrier()
        pltpu.sync_copy(acc.at[pl.ds(s * rpt, rpt)],
                        out_hbm.at[c, pl.ds(s * rpt, rpt)])

    return deg_kernel(dst2d, ones, zeros)


def _sc_aggregate(u, src2d, dst2d):
    n, d = u.shape
    nchunks = src2d.shape[0]
    cpt = nchunks // (NC * NS)
    rpt = NPAD // NS
    zeros = jnp.zeros((rpt, d), jnp.float32)
    mesh = plsc.VectorSubcoreMesh(core_axis_name="c", subcore_axis_name="s")

    @functools.partial(
        pl.kernel,
        out_type=jax.ShapeDtypeStruct((NC, NPAD, d), jnp.float32),
        mesh=mesh,
        scratch_types=[
            pltpu.VMEM((cpt, K), jnp.int32),
            pltpu.VMEM((cpt, K), jnp.int32),
            pltpu.VMEM((K, d), jnp.float32),
            pltpu.VMEM_SHARED((NPAD, d), jnp.float32),
            pltpu.SemaphoreType.DMA,
        ],
    )
    def agg_kernel(u_hbm, src_hbm, dst_hbm, z_hbm, out_hbm,
                   src_v, dst_v, rows0, acc, gsem):
        c = lax.axis_index("c")
        s = lax.axis_index("s")
        wid = c * NS + s
        pltpu.sync_copy(src_hbm.at[pl.ds(wid * cpt, cpt)], src_v)
        pltpu.sync_copy(dst_hbm.at[pl.ds(wid * cpt, cpt)], dst_v)
        pltpu.sync_copy(z_hbm, acc.at[pl.ds(s * rpt, rpt)])
        plsc.subcore_barrier()

        def body(j, carry):
            pltpu.async_copy(u_hbm.at[src_v.at[j]], rows0, gsem).wait()
            pltpu.sync_copy(rows0, acc.at[dst_v.at[j]], add=True)
            return carry

        lax.fori_loop(0, cpt, body, 0, unroll=False)
        plsc.subcore_barrier()
        pltpu.sync_copy(acc.at[pl.ds(s * rpt, rpt)],
                        out_hbm.at[c, pl.ds(s * rpt, rpt)])

    return agg_kernel(u, src2d, dst2d, zeros)


def _sc_aggregate2(ua, ub, src2d, dst2d):
    n, d = ua.shape
    nchunks = src2d.shape[0]
    cpt = nchunks // (NC * NS)
    rpt = NPAD // NS
    zeros = jnp.zeros((rpt, d), jnp.float32)
    mesh = plsc.VectorSubcoreMesh(core_axis_name="c", subcore_axis_name="s")
    out_t = jax.ShapeDtypeStruct((NC, NPAD, d), jnp.float32)

    @functools.partial(
        pl.kernel,
        out_type=(out_t, out_t),
        mesh=mesh,
        scratch_types=[
            pltpu.VMEM((cpt, K), jnp.int32),
            pltpu.VMEM((cpt, K), jnp.int32),
            pltpu.VMEM((K, d), jnp.float32),
            pltpu.VMEM_SHARED((NPAD, d), jnp.float32),
            pltpu.SemaphoreType.DMA,
        ],
    )
    def agg2_kernel(ua_hbm, ub_hbm, src_hbm, dst_hbm, z_hbm,
                    outa_hbm, outb_hbm, src_v, dst_v, rows0, acc, gsem):
        c = lax.axis_index("c")
        s = lax.axis_index("s")
        wid = c * NS + s
        pltpu.sync_copy(src_hbm.at[pl.ds(wid * cpt, cpt)], src_v)
        pltpu.sync_copy(dst_hbm.at[pl.ds(wid * cpt, cpt)], dst_v)

        for u_hbm, out_hbm in ((ua_hbm, outa_hbm), (ub_hbm, outb_hbm)):
            pltpu.sync_copy(z_hbm, acc.at[pl.ds(s * rpt, rpt)])
            plsc.subcore_barrier()

            def body(j, carry):
                pltpu.async_copy(u_hbm.at[src_v.at[j]], rows0, gsem).wait()
                pltpu.sync_copy(rows0, acc.at[dst_v.at[j]], add=True)
                return carry

            lax.fori_loop(0, cpt, body, 0, unroll=False)
            plsc.subcore_barrier()
            pltpu.sync_copy(acc.at[pl.ds(s * rpt, rpt)],
                            out_hbm.at[c, pl.ds(s * rpt, rpt)])
            plsc.subcore_barrier()

    return agg2_kernel(ua, ub, src2d, dst2d, zeros)


_ROWS = 2000


def _row_spec(d):
    return pl.BlockSpec((_ROWS, d), lambda i: (i, 0))


def _full_spec(r, c):
    return pl.BlockSpec((r, c), lambda i: (0, 0))


def _dinv(d0_ref, d1_ref):
    deg = d0_ref[...][:, :1] + d1_ref[...][:, :1] + 1.0
    return lax.rsqrt(deg)


def _tc_lin1(x, W1):
    n, din = x.shape
    d1 = W1.shape[1]
    h = d1 // 2

    def body(x_r, w_r, oa_r, ob_r):
        t = jnp.dot(x_r[...], w_r[...], preferred_element_type=jnp.float32)
        oa_r[...] = t[:, :h]
        ob_r[...] = t[:, h:]

    return pl.pallas_call(
        body,
        grid=(n // _ROWS,),
        in_specs=[_row_spec(din), _full_spec(din, d1)],
        out_specs=[_row_spec(h), _row_spec(h)],
        out_shape=[jax.ShapeDtypeStruct((n, h), jnp.float32),
                   jax.ShapeDtypeStruct((n, h), jnp.float32)],
    )(x, W1)


def _tc_scale2(ta, tb, deg0, deg1):
    n, h = ta.shape

    def body(ta_r, tb_r, d0_r, d1_r, oa_r, ob_r):
        dinv = _dinv(d0_r, d1_r)
        oa_r[...] = dinv * ta_r[...]
        ob_r[...] = dinv * tb_r[...]

    return pl.pallas_call(
        body,
        grid=(n // _ROWS,),
        in_specs=[_row_spec(h), _row_spec(h), _row_spec(DW), _row_spec(DW)],
        out_specs=[_row_spec(h), _row_spec(h)],
        out_shape=[jax.ShapeDtypeStruct((n, h), jnp.float32),
                   jax.ShapeDtypeStruct((n, h), jnp.float32)],
    )(ta, tb, deg0, deg1)


def _tc_layer1(sa, sb, u1a, u1b, deg0, deg1, b1, W2):
    n, h = u1a.shape
    d2 = W2.shape[1]

    def body(sa0_r, sa1_r, sb0_r, sb1_r, ua_r, ub_r, d0_r, d1_r,
             b1_r, w2_r, o_r):
        dinv = _dinv(d0_r, d1_r)
        ga = dinv * (sa0_r[...] + sa1_r[...] + ua_r[...])
        gb = dinv * (sb0_r[...] + sb1_r[...] + ub_r[...])
        g = jnp.concatenate([ga, gb], axis=1)
        y = jnp.maximum(g + b1_r[...], 0.0)
        o_r[...] = dinv * jnp.dot(y, w2_r[...],
                                  preferred_element_type=jnp.float32)

    return pl.pallas_call(
        body,
        grid=(n // _ROWS,),
        in_specs=[_row_spec(h), _row_spec(h), _row_spec(h), _row_spec(h),
                  _row_spec(h), _row_spec(h),
                  _row_spec(DW), _row_spec(DW),
                  _full_spec(1, 2 * h), _full_spec(2 * h, d2)],
        out_specs=_row_spec(d2),
        out_shape=jax.ShapeDtypeStruct((n, d2), jnp.float32),
    )(sa[0], sa[1], sb[0], sb[1], u1a, u1b, deg0, deg1, b1, W2)


def _tc_layer2(sp, u2, deg0, deg1, b2, W3p):
    n, din = u2.shape
    d3 = W3p.shape[1]

    def body(s0_r, s1_r, u_r, d0_r, d1_r, b2_r, w3_r, o_r):
        dinv = _dinv(d0_r, d1_r)
        g = dinv * (s0_r[...] + s1_r[...] + u_r[...])
        y = jnp.maximum(g + b2_r[...], 0.0)
        o_r[...] = dinv * jnp.dot(y, w3_r[...],
                                  preferred_element_type=jnp.float32)

    return pl.pallas_call(
        body,
        grid=(n // _ROWS,),
        in_specs=[_row_spec(din), _row_spec(din), _row_spec(din),
                  _row_spec(DW), _row_spec(DW),
                  _full_spec(1, din), _full_spec(din, d3)],
        out_specs=_row_spec(d3),
        out_shape=jax.ShapeDtypeStruct((n, d3), jnp.float32),
    )(sp[0], sp[1], u2, deg0, deg1, b2, W3p)


def _tc_layer3(sp, u3, deg0, deg1, b3p, Wl, bl, d3):
    n, din = u3.shape

    def body(s0_r, s1_r, u_r, d0_r, d1_r, b3_r, wl_r, bl_r, o_r):
        dinv = _dinv(d0_r, d1_r)
        g = dinv * (s0_r[...] + s1_r[...] + u_r[...])
        y = jnp.maximum(g + b3_r[...], 0.0)
        o_r[...] = jnp.dot(y[:, :d3], wl_r[...],
                           preferred_element_type=jnp.float32) + bl_r[...]

    return pl.pallas_call(
        body,
        grid=(n // _ROWS,),
        in_specs=[_row_spec(din), _row_spec(din), _row_spec(din),
                  _row_spec(DW), _row_spec(DW),
                  _full_spec(1, din), _full_spec(d3, 1), _full_spec(1, 1)],
        out_specs=_row_spec(1),
        out_shape=jax.ShapeDtypeStruct((n, 1), jnp.float32),
    )(sp[0], sp[1], u3, deg0, deg1, b3p, Wl, bl)


@jax.jit
def kernel(x, edge_index, W1, b1, W2, b2, W3, b3, Wl, bl):
    n = x.shape[0]
    e = edge_index.shape[1]
    cpt = -(-e // (NC * NS * K))
    cpt = (cpt + 7) // 8 * 8
    epad = NC * NS * cpt * K - e
    src2d = jnp.concatenate(
        [edge_index[0], jnp.zeros((epad,), edge_index.dtype)]).reshape(-1, K)
    dst2d = jnp.concatenate(
        [edge_index[1], jnp.full((epad,), n, edge_index.dtype)]).reshape(-1, K)

    d3 = W3.shape[1]
    W3p = jnp.pad(W3, ((0, 0), (0, DW - d3)))
    b3p = jnp.pad(b3, (0, DW - d3))

    t1a, t1b = _tc_lin1(x, W1)
    degp = _sc_degree(dst2d, n)
    deg0, deg1 = degp[0], degp[1]

    u1a, u1b = _tc_scale2(t1a, t1b, deg0, deg1)
    sa, sb = _sc_aggregate2(u1a, u1b, src2d, dst2d)
    u2 = _tc_layer1(sa, sb, u1a, u1b, deg0, deg1,
                    b1.reshape(1, -1), W2)
    s2 = _sc_aggregate(u2, src2d, dst2d)
    u3 = _tc_layer2(s2, u2, deg0, deg1,
                    b2.reshape(1, -1), W3p)
    s3 = _sc_aggregate(u3, src2d, dst2d)
    out = _tc_layer3(s3, u3, deg0, deg1,
                     b3p.reshape(1, -1), Wl,
                     bl.reshape(1, 1), d3)
    return out


# speedup vs baseline: 2.5968x; 2.5968x over previous
import functools

import jax
import jax.numpy as jnp
from jax import lax
from jax.experimental import pallas as pl
from jax.experimental.pallas import tpu as pltpu
from jax.experimental.pallas import tpu_sc as plsc

NC = 2
NS = 16
K = 125
NPAD = 10240
DW = 128


def _sc_degree(dst2d, n):
    nchunks = dst2d.shape[0]
    cpt = nchunks // (NC * NS)
    rpt = NPAD // NS
    ones = jnp.ones((K, DW), jnp.float32)
    zeros = jnp.zeros((rpt, DW), jnp.float32)
    mesh = plsc.VectorSubcoreMesh(core_axis_name="c", subcore_axis_name="s")

    @functools.partial(
        pl.kernel,
        out_type=jax.ShapeDtypeStruct((NC, NPAD, DW), jnp.float32),
        mesh=mesh,
        scratch_types=[
            pltpu.VMEM((cpt, K), jnp.int32),
            pltpu.VMEM((K, DW), jnp.float32),
            pltpu.VMEM_SHARED((NPAD, DW), jnp.float32),
        ],
    )
    def deg_kernel(dst_hbm, ones_hbm, z_hbm, out_hbm, dst_v, ones_v, acc):
        c = lax.axis_index("c")
        s = lax.axis_index("s")
        wid = c * NS + s
        pltpu.sync_copy(z_hbm, acc.at[pl.ds(s * rpt, rpt)])
        pltpu.sync_copy(dst_hbm.at[pl.ds(wid * cpt, cpt)], dst_v)
        pltpu.sync_copy(ones_hbm, ones_v)
        plsc.subcore_barrier()

        def body(j, carry):
            pltpu.sync_copy(ones_v, acc.at[dst_v.at[j]], add=True)
            return carry

        lax.fori_loop(0, cpt, body, 0, unroll=False)
        plsc.subcore_barrier()
        pltpu.sync_copy(acc.at[pl.ds(s * rpt, rpt)],
                        out_hbm.at[c, pl.ds(s * rpt, rpt)])

    return deg_kernel(dst2d, ones, zeros)


def _sc_aggregate(u, src2d, dst2d):
    n, d = u.shape
    nchunks = src2d.shape[0]
    cpt = nchunks // (NC * NS)
    rpt = NPAD // NS
    zeros = jnp.zeros((rpt, d), jnp.float32)
    mesh = plsc.VectorSubcoreMesh(core_axis_name="c", subcore_axis_name="s")

    @functools.partial(
        pl.kernel,
        out_type=jax.ShapeDtypeStruct((NC, NPAD, d), jnp.float32),
        mesh=mesh,
        scratch_types=[
            pltpu.VMEM((cpt, K), jnp.int32),
            pltpu.VMEM((cpt, K), jnp.int32),
            pltpu.VMEM((K, d), jnp.float32),
            pltpu.VMEM_SHARED((NPAD, d), jnp.float32),
            pltpu.SemaphoreType.DMA,
        ],
    )
    def agg_kernel(u_hbm, src_hbm, dst_hbm, z_hbm, out_hbm,
                   src_v, dst_v, rows0, acc, gsem):
        c = lax.axis_index("c")
        s = lax.axis_index("s")
        wid = c * NS + s
        pltpu.sync_copy(src_hbm.at[pl.ds(wid * cpt, cpt)], src_v)
        pltpu.sync_copy(dst_hbm.at[pl.ds(wid * cpt, cpt)], dst_v)
        pltpu.sync_copy(z_hbm, acc.at[pl.ds(s * rpt, rpt)])
        plsc.subcore_barrier()

        def body(j, carry):
            pltpu.async_copy(u_hbm.at[src_v.at[j]], rows0, gsem).wait()
            pltpu.sync_copy(rows0, acc.at[dst_v.at[j]], add=True)
            return carry

        lax.fori_loop(0, cpt, body, 0, unroll=False)
        plsc.subcore_barrier()
        pltpu.sync_copy(acc.at[pl.ds(s * rpt, rpt)],
                        out_hbm.at[c, pl.ds(s * rpt, rpt)])

    return agg_kernel(u, src2d, dst2d, zeros)


def _sc_aggregate2(ua, ub, src2d, dst2d):
    n, d = ua.shape
    nchunks = src2d.shape[0]
    cpt = nchunks // (NC * NS)
    rpt = NPAD // NS
    zeros = jnp.zeros((rpt, d), jnp.float32)
    mesh = plsc.VectorSubcoreMesh(core_axis_name="c", subcore_axis_name="s")
    out_t = jax.ShapeDtypeStruct((NC, NPAD, d), jnp.float32)

    @functools.partial(
        pl.kernel,
        out_type=(out_t, out_t),
        mesh=mesh,
        scratch_types=[
            pltpu.VMEM((cpt, K), jnp.int32),
            pltpu.VMEM((cpt, K), jnp.int32),
            pltpu.VMEM((K, d), jnp.float32),
            pltpu.VMEM_SHARED((NPAD, d), jnp.float32),
            pltpu.SemaphoreType.DMA,
        ],
    )
    def agg2_kernel(ua_hbm, ub_hbm, src_hbm, dst_hbm, z_hbm,
                    outa_hbm, outb_hbm, src_v, dst_v, rows0, acc, gsem):
        c = lax.axis_index("c")
        s = lax.axis_index("s")
        wid = c * NS + s
        pltpu.sync_copy(src_hbm.at[pl.ds(wid * cpt, cpt)], src_v)
        pltpu.sync_copy(dst_hbm.at[pl.ds(wid * cpt, cpt)], dst_v)

        for u_hbm, out_hbm in ((ua_hbm, outa_hbm), (ub_hbm, outb_hbm)):
            pltpu.sync_copy(z_hbm, acc.at[pl.ds(s * rpt, rpt)])
            plsc.subcore_barrier()

            def body(j, carry):
                pltpu.async_copy(u_hbm.at[src_v.at[j]], rows0, gsem).wait()
                pltpu.sync_copy(rows0, acc.at[dst_v.at[j]], add=True)
                return carry

            lax.fori_loop(0, cpt, body, 0, unroll=False)
            plsc.subcore_barrier()
            pltpu.sync_copy(acc.at[pl.ds(s * rpt, rpt)],
                            out_hbm.at[c, pl.ds(s * rpt, rpt)])
            plsc.subcore_barrier()

    return agg2_kernel(ua, ub, src2d, dst2d, zeros)


_ROWS = 2000


def _row_spec(d):
    return pl.BlockSpec((_ROWS, d), lambda i: (i, 0))


def _full_spec(r, c):
    return pl.BlockSpec((r, c), lambda i: (0, 0))


def _dinv(d0_ref, d1_ref):
    deg = d0_ref[...][:, :1] + d1_ref[...][:, :1] + 1.0
    return lax.rsqrt(deg)


def _tc_lin1(x, W1):
    n, din = x.shape
    d1 = W1.shape[1]
    h = d1 // 2

    def body(x_r, w_r, oa_r, ob_r):
        t = jnp.dot(x_r[...], w_r[...], preferred_element_type=jnp.float32)
        oa_r[...] = t[:, :h]
        ob_r[...] = t[:, h:]

    return pl.pallas_call(
        body,
        grid=(n // _ROWS,),
        in_specs=[_row_spec(din), _full_spec(din, d1)],
        out_specs=[_row_spec(h), _row_spec(h)],
        out_shape=[jax.ShapeDtypeStruct((n, h), jnp.float32),
                   jax.ShapeDtypeStruct((n, h), jnp.float32)],
    )(x, W1)


def _tc_scale2(ta, tb, deg0, deg1):
    n, h = ta.shape

    def body(ta_r, tb_r, d0_r, d1_r, oa_r, ob_r):
        dinv = _dinv(d0_r, d1_r)
        oa_r[...] = dinv * ta_r[...]
        ob_r[...] = dinv * tb_r[...]

    return pl.pallas_call(
        body,
        grid=(n // _ROWS,),
        in_specs=[_row_spec(h), _row_spec(h), _row_spec(DW), _row_spec(DW)],
        out_specs=[_row_spec(h), _row_spec(h)],
        out_shape=[jax.ShapeDtypeStruct((n, h), jnp.float32),
                   jax.ShapeDtypeStruct((n, h), jnp.float32)],
    )(ta, tb, deg0, deg1)


def _tc_layer1(sa, sb, u1a, u1b, deg0, deg1, b1, W2):
    n, h = u1a.shape
    d2 = W2.shape[1]

    def body(sa0_r, sa1_r, sb0_r, sb1_r, ua_r, ub_r, d0_r, d1_r,
             b1_r, w2_r, o_r):
        dinv = _dinv(d0_r, d1_r)
        ga = dinv * (sa0_r[...] + sa1_r[...] + ua_r[...])
        gb = dinv * (sb0_r[...] + sb1_r[...] + ub_r[...])
        g = jnp.concatenate([ga, gb], axis=1)
        y = jnp.maximum(g + b1_r[...], 0.0)
        o_r[...] = dinv * jnp.dot(y, w2_r[...],
                                  preferred_element_type=jnp.float32)

    return pl.pallas_call(
        body,
        grid=(n // _ROWS,),
        in_specs=[_row_spec(h), _row_spec(h), _row_spec(h), _row_spec(h),
                  _row_spec(h), _row_spec(h),
                  _row_spec(DW), _row_spec(DW),
                  _full_spec(1, 2 * h), _full_spec(2 * h, d2)],
        out_specs=_row_spec(d2),
        out_shape=jax.ShapeDtypeStruct((n, d2), jnp.float32),
    )(sa[0], sa[1], sb[0], sb[1], u1a, u1b, deg0, deg1, b1, W2)


def _tc_layer2(sp, u2, deg0, deg1, b2, W3p):
    n, din = u2.shape
    d3 = W3p.shape[1]

    def body(s0_r, s1_r, u_r, d0_r, d1_r, b2_r, w3_r, o_r):
        dinv = _dinv(d0_r, d1_r)
        g = dinv * (s0_r[...] + s1_r[...] + u_r[...])
        y = jnp.maximum(g + b2_r[...], 0.0)
        o_r[...] = dinv * jnp.dot(y, w3_r[...],
                                  preferred_element_type=jnp.float32)

    return pl.pallas_call(
        body,
        grid=(n // _ROWS,),
        in_specs=[_row_spec(din), _row_spec(din), _row_spec(din),
                  _row_spec(DW), _row_spec(DW),
                  _full_spec(1, din), _full_spec(din, d3)],
        out_specs=_row_spec(d3),
        out_shape=jax.ShapeDtypeStruct((n, d3), jnp.float32),
    )(sp[0], sp[1], u2, deg0, deg1, b2, W3p)


def _tc_layer3(sp, u3, deg0, deg1, b3p, Wl, bl, d3):
    n, din = u3.shape

    def body(s0_r, s1_r, u_r, d0_r, d1_r, b3_r, wl_r, bl_r, o_r):
        dinv = _dinv(d0_r, d1_r)
        g = dinv * (s0_r[...] + s1_r[...] + u_r[...])
        y = jnp.maximum(g + b3_r[...], 0.0)
        o_r[...] = jnp.dot(y[:, :d3], wl_r[...],
                           preferred_element_type=jnp.float32) + bl_r[...]

    return pl.pallas_call(
        body,
        grid=(n // _ROWS,),
        in_specs=[_row_spec(din), _row_spec(din), _row_spec(din),
                  _row_spec(DW), _row_spec(DW),
                  _full_spec(1, din), _full_spec(d3, 1), _full_spec(1, 1)],
        out_specs=_row_spec(1),
        out_shape=jax.ShapeDtypeStruct((n, 1), jnp.float32),
    )(sp[0], sp[1], u3, deg0, deg1, b3p, Wl, bl)


@jax.jit
def kernel(x, edge_index, W1, b1, W2, b2, W3, b3, Wl, bl):
    n = x.shape[0]
    e = edge_index.shape[1]
    cpt = -(-e // (NC * NS * K))
    cpt = (cpt + 7) // 8 * 8
    epad = NC * NS * cpt * K - e
    if epad:
        pad_dst = n + (jnp.arange(epad, dtype=edge_index.dtype)
                       % (NPAD - n))
        src_flat = jnp.concatenate(
            [edge_index[0], jnp.zeros((epad,), edge_index.dtype)])
        dst_flat = jnp.concatenate([edge_index[1], pad_dst])
    else:
        src_flat, dst_flat = edge_index[0], edge_index[1]
    src2d = src_flat.reshape(-1, K)
    dst2d = dst_flat.reshape(-1, K)

    d3 = W3.shape[1]
    W3p = jnp.pad(W3, ((0, 0), (0, DW - d3)))
    b3p = jnp.pad(b3, (0, DW - d3))

    t1a, t1b = _tc_lin1(x, W1)
    degp = _sc_degree(dst2d, n)
    deg0, deg1 = degp[0], degp[1]

    u1a, u1b = _tc_scale2(t1a, t1b, deg0, deg1)
    sa, sb = _sc_aggregate2(u1a, u1b, src2d, dst2d)
    u2 = _tc_layer1(sa, sb, u1a, u1b, deg0, deg1,
                    b1.reshape(1, -1), W2)
    s2 = _sc_aggregate(u2, src2d, dst2d)
    u3 = _tc_layer2(s2, u2, deg0, deg1,
                    b2.reshape(1, -1), W3p)
    s3 = _sc_aggregate(u3, src2d, dst2d)
    out = _tc_layer3(s3, u3, deg0, deg1,
                     b3p.reshape(1, -1), Wl,
                     bl.reshape(1, 1), d3)
    return out
